# baseline (speedup 1.0000x reference)
import jax
import jax.numpy as jnp
from jax.experimental import pallas as pl
from jax.experimental.pallas import tpu as pltpu

_BK = 128


def _gcnn_kernel(adj_ref, x_ref, w_ref, b_ref, out_ref, ab_ref, cs_ref):
    k = pl.program_id(0)
    num_blocks = pl.num_programs(0) - 1

    @pl.when(k == 0)
    def _init():
        cs_ref[...] = jnp.zeros_like(cs_ref)

    @pl.when(k < num_blocks)
    def _accumulate():
        blk = adj_ref[...].astype(jnp.float32)
        cs_ref[...] += jnp.sum(blk, axis=0, keepdims=True)
        ab_ref[pl.ds(k * _BK, _BK), :] = blk.astype(jnp.bfloat16)

    @pl.when(k == num_blocks)
    def _finalize():
        d = jax.lax.rsqrt(cs_ref[...] + 1.0)
        dc = d.reshape(-1, 1)
        xw = jnp.dot(x_ref[...], w_ref[...], preferred_element_type=jnp.float32)
        y = xw * dc
        z = jax.lax.dot_general(ab_ref[...], y.astype(jnp.bfloat16),
                                (((0,), (0,)), ((), ())),
                                preferred_element_type=jnp.float32)
        out_ref[...] = (z + y) * dc + b_ref[...]


def kernel(batch_inputs, batch_graph, W, b):
    n, f = batch_inputs.shape
    fo = W.shape[1]
    num_blocks = n // _BK
    return pl.pallas_call(
        _gcnn_kernel,
        grid=(num_blocks + 1,),
        in_specs=[
            pl.BlockSpec((_BK, n), lambda k: (jnp.minimum(k, num_blocks - 1), 0)),
            pl.BlockSpec((n, f), lambda k: (0, 0)),
            pl.BlockSpec((f, fo), lambda k: (0, 0)),
            pl.BlockSpec((1, fo), lambda k: (0, 0)),
        ],
        out_specs=pl.BlockSpec((n, fo), lambda k: (0, 0)),
        scratch_shapes=[
            pltpu.VMEM((n, n), jnp.bfloat16),
            pltpu.VMEM((1, n), jnp.float32),
        ],
        out_shape=jax.ShapeDtypeStruct((n, fo), batch_inputs.dtype),
    )(batch_graph, batch_inputs, W, b.reshape(1, -1))


# speedup vs baseline: 1.6178x; 1.6178x over previous
import jax
import jax.numpy as jnp
from jax.experimental import pallas as pl


def _gcnn_kernel(adj_ref, x_ref, w_ref, b_ref, out_ref):
    ai = adj_ref[...]
    deg = jnp.sum(ai, axis=0, keepdims=True)
    d = jax.lax.rsqrt(deg.astype(jnp.float32) + 1.0)
    dc = d.reshape(-1, 1)
    xw = jnp.dot(x_ref[...], w_ref[...], preferred_element_type=jnp.float32)
    y = xw * dc
    z = jax.lax.dot_general(ai.astype(jnp.bfloat16), y.astype(jnp.bfloat16),
                            (((0,), (0,)), ((), ())),
                            preferred_element_type=jnp.float32)
    out_ref[...] = (z + y) * dc + b_ref[...]


def kernel(batch_inputs, batch_graph, W, b):
    n, f = batch_inputs.shape
    return pl.pallas_call(
        _gcnn_kernel,
        out_shape=jax.ShapeDtypeStruct((n, W.shape[1]), batch_inputs.dtype),
    )(batch_graph, batch_inputs, W, b.reshape(1, -1))
